# Initial kernel scaffold; baseline (speedup 1.0000x reference)
import jax
import jax.numpy as jnp
from jax.experimental import pallas as pl


def kernel(x, table, W1, b1, W2, b2, W3, b3):
    raise NotImplementedError("write your pallas kernel here")


# speedup vs baseline: 14.0545x; 14.0545x over previous
import functools

import jax
import jax.numpy as jnp
from jax import lax
from jax.experimental import pallas as pl
from jax.experimental.pallas import tpu as pltpu
from jax.experimental.pallas import tpu_sc as plsc

VOCAB = 1000000
EMB = 32
WIN = 20
H1 = 128
H2 = 64
NCLS = 9
BATCH = 16384
NIDX = BATCH * WIN

NC = 2
NS = 16
NW = NC * NS
ROWS_PER_W = NIDX // NW
G = 128
CHUNK = 2048
G_PER_CHUNK = CHUNK // G
N_CHUNK = ROWS_PER_W // CHUNK

_sc_mesh = plsc.VectorSubcoreMesh(
    core_axis_name="c", subcore_axis_name="s", num_cores=NC, num_subcores=NS
)


@functools.partial(
    pl.kernel,
    out_type=jax.ShapeDtypeStruct((NIDX, EMB), jnp.float32),
    mesh=_sc_mesh,
    scratch_types=[
        pltpu.VMEM((ROWS_PER_W // G, G), jnp.int32),
        pltpu.VMEM((CHUNK, EMB), jnp.float32),
        pltpu.SemaphoreType.DMA,
    ],
    compiler_params=pltpu.CompilerParams(use_tc_tiling_on_sc=False),
)
def _sc_gather(table_hbm, idx_hbm, out_hbm, idx_v, rows_v, sem):
    wid = lax.axis_index("s") * NC + lax.axis_index("c")
    pltpu.sync_copy(idx_hbm.at[pl.ds(wid * (ROWS_PER_W // G), ROWS_PER_W // G)], idx_v)
    base = wid * ROWS_PER_W

    def chunk_body(ci, _):
        copies = []
        for j in range(G_PER_CHUNK):
            cp = pltpu.make_async_copy(
                table_hbm.at[idx_v.at[ci * G_PER_CHUNK + j]],
                rows_v.at[pl.ds(j * G, G)],
                sem,
            )
            cp.start()
            copies.append(cp)
        for cp in copies:
            cp.wait()
        pltpu.sync_copy(rows_v, out_hbm.at[pl.ds(base + ci * CHUNK, CHUNK)])
        return ()

    lax.fori_loop(0, N_CHUNK, chunk_body, (), unroll=False)


BLK = 1024


def _mlp_body(x_ref, w1_ref, b1_ref, w2_ref, b2_ref, w3_ref, b3_ref, o_ref):
    h = jnp.dot(x_ref[...], w1_ref[...], preferred_element_type=jnp.float32)
    h = jnp.maximum(h + b1_ref[...], 0.0)
    h = jnp.dot(h, w2_ref[...], preferred_element_type=jnp.float32)
    h = jnp.maximum(h + b2_ref[...], 0.0)
    o_ref[...] = jnp.dot(h, w3_ref[...], preferred_element_type=jnp.float32) + b3_ref[...]


def _mlp(h, W1, b1, W2, b2, W3, b3):
    in_dim = WIN * EMB
    return pl.pallas_call(
        _mlp_body,
        grid=(BATCH // BLK,),
        in_specs=[
            pl.BlockSpec((BLK, in_dim), lambda i: (i, 0)),
            pl.BlockSpec((in_dim, H1), lambda i: (0, 0)),
            pl.BlockSpec((1, H1), lambda i: (0, 0)),
            pl.BlockSpec((H1, H2), lambda i: (0, 0)),
            pl.BlockSpec((1, H2), lambda i: (0, 0)),
            pl.BlockSpec((H2, NCLS), lambda i: (0, 0)),
            pl.BlockSpec((1, NCLS), lambda i: (0, 0)),
        ],
        out_specs=pl.BlockSpec((BLK, NCLS), lambda i: (i, 0)),
        out_shape=jax.ShapeDtypeStruct((BATCH, NCLS), jnp.float32),
    )(h, W1, b1, W2, b2, W3, b3)


@jax.jit
def kernel(x, table, W1, b1, W2, b2, W3, b3):
    idx2d = x.astype(jnp.int32).reshape(NIDX // G, G)
    embeds = _sc_gather(table, idx2d)
    h = embeds.reshape(BATCH, WIN * EMB)
    return _mlp(
        h,
        W1,
        b1.reshape(1, H1),
        W2,
        b2.reshape(1, H2),
        W3,
        b3.reshape(1, NCLS),
    )


# speedup vs baseline: 21.2808x; 1.5142x over previous
import functools

import jax
import jax.numpy as jnp
from jax import lax
from jax.experimental import pallas as pl
from jax.experimental.pallas import tpu as pltpu
from jax.experimental.pallas import tpu_sc as plsc

VOCAB = 1000000
EMB = 32
WIN = 20
H1 = 128
H2 = 64
NCLS = 9
BATCH = 16384
NIDX = BATCH * WIN

NC = 2
NS = 16
NW = NC * NS
ROWS_PER_W = NIDX // NW
G = 128
CHUNK = 2048
G_PER_CHUNK = CHUNK // G
N_CHUNK = ROWS_PER_W // CHUNK

_sc_mesh = plsc.VectorSubcoreMesh(
    core_axis_name="c", subcore_axis_name="s", num_cores=NC, num_subcores=NS
)


@functools.partial(
    pl.kernel,
    out_type=jax.ShapeDtypeStruct((NIDX, EMB), jnp.float32),
    mesh=_sc_mesh,
    scratch_types=[
        pltpu.VMEM((ROWS_PER_W // G, G), jnp.int32),
        pltpu.VMEM((CHUNK, EMB), jnp.float32),
        pltpu.SemaphoreType.DMA,
    ],
    compiler_params=pltpu.CompilerParams(use_tc_tiling_on_sc=False),
)
def _sc_gather(table_hbm, idx_hbm, out_hbm, idx_v, rows_v, sem):
    wid = lax.axis_index("s") * NC + lax.axis_index("c")
    pltpu.sync_copy(idx_hbm.at[pl.ds(wid * (ROWS_PER_W // G), ROWS_PER_W // G)], idx_v)

    def remap_body(t, _):
        v = idx_v[t // 8, pl.ds((t % 8) * 16, 16)]
        j = jnp.bitwise_and(v, QSPLIT - 1) * 4 + lax.shift_right_logical(v, 18)
        idx_v[t // 8, pl.ds((t % 8) * 16, 16)] = j
        return ()

    lax.fori_loop(0, (ROWS_PER_W // G) * (G // 16), remap_body, (), unroll=False)
    base = wid * ROWS_PER_W

    def chunk_body(ci, _):
        copies = []
        for j in range(G_PER_CHUNK):
            cp = pltpu.make_async_copy(
                table_hbm.at[idx_v.at[ci * G_PER_CHUNK + j]],
                rows_v.at[pl.ds(j * G, G)],
                sem,
            )
            cp.start()
            copies.append(cp)
        for cp in copies:
            cp.wait()
        pltpu.sync_copy(rows_v, out_hbm.at[pl.ds(base + ci * CHUNK, CHUNK)])
        return ()

    lax.fori_loop(0, N_CHUNK, chunk_body, (), unroll=False)


QSPLIT = 1 << 18
VPAD = 4 * QSPLIT
RB = 2048


def _transpose_body(t0_ref, t1_ref, t2_ref, t3_ref, o_ref):
    parts = [jnp.transpose(t[...]) for t in (t0_ref, t1_ref, t2_ref, t3_ref)]
    o_ref[...] = jnp.concatenate(parts, axis=1)


def _relayout_table(tableT):
    grid = QSPLIT // RB

    last_blk = (VOCAB - 1) // RB

    def mk(q):
        return pl.BlockSpec(
            (EMB, RB),
            lambda i, q=q: (0, jnp.minimum((QSPLIT // RB) * q + i, last_blk)),
        )

    out = pl.pallas_call(
        _transpose_body,
        grid=(grid,),
        in_specs=[mk(0), mk(1), mk(2), mk(3)],
        out_specs=pl.BlockSpec((RB, 4 * EMB), lambda i: (i, 0)),
        out_shape=jax.ShapeDtypeStruct((QSPLIT, 4 * EMB), jnp.float32),
    )(tableT, tableT, tableT, tableT)
    return out.reshape(VPAD, EMB)


BLK = 1024


def _mlp_body(x_ref, w1_ref, b1_ref, w2_ref, b2_ref, w3_ref, b3_ref, o_ref):
    h = jnp.dot(x_ref[...], w1_ref[...], preferred_element_type=jnp.float32)
    h = jnp.maximum(h + b1_ref[...], 0.0)
    h = jnp.dot(h, w2_ref[...], preferred_element_type=jnp.float32)
    h = jnp.maximum(h + b2_ref[...], 0.0)
    o_ref[...] = jnp.dot(h, w3_ref[...], preferred_element_type=jnp.float32) + b3_ref[...]


def _mlp(h, W1, b1, W2, b2, W3, b3):
    in_dim = WIN * EMB
    return pl.pallas_call(
        _mlp_body,
        grid=(BATCH // BLK,),
        in_specs=[
            pl.BlockSpec((BLK, in_dim), lambda i: (i, 0)),
            pl.BlockSpec((in_dim, H1), lambda i: (0, 0)),
            pl.BlockSpec((1, H1), lambda i: (0, 0)),
            pl.BlockSpec((H1, H2), lambda i: (0, 0)),
            pl.BlockSpec((1, H2), lambda i: (0, 0)),
            pl.BlockSpec((H2, NCLS), lambda i: (0, 0)),
            pl.BlockSpec((1, NCLS), lambda i: (0, 0)),
        ],
        out_specs=pl.BlockSpec((BLK, NCLS), lambda i: (i, 0)),
        out_shape=jax.ShapeDtypeStruct((BATCH, NCLS), jnp.float32),
    )(h, W1, b1, W2, b2, W3, b3)


@jax.jit
def kernel(x, table, W1, b1, W2, b2, W3, b3):
    idx2d = x.astype(jnp.int32).reshape(NIDX // G, G)
    table_lin = _relayout_table(table.T)
    embeds = _sc_gather(table_lin, idx2d)
    h = embeds.reshape(BATCH, WIN * EMB)
    return _mlp(
        h,
        W1,
        b1.reshape(1, H1),
        W2,
        b2.reshape(1, H2),
        W3,
        b3.reshape(1, NCLS),
    )


# speedup vs baseline: 48.7231x; 2.2895x over previous
import functools

import jax
import jax.numpy as jnp
from jax import lax
from jax.experimental import pallas as pl
from jax.experimental.pallas import tpu as pltpu
from jax.experimental.pallas import tpu_sc as plsc

VOCAB = 1000000
EMB = 32
WIN = 20
H1 = 128
H2 = 64
NCLS = 9
BATCH = 16384
NIDX = BATCH * WIN

NC = 2
NS = 16
NW = NC * NS
ROWS_PER_W = NIDX // NW
G = 128
CHUNK = 2048
G_PER_CHUNK = CHUNK // G
N_CHUNK = ROWS_PER_W // CHUNK

_sc_mesh = plsc.VectorSubcoreMesh(
    core_axis_name="c", subcore_axis_name="s", num_cores=NC, num_subcores=NS
)


@functools.partial(
    pl.kernel,
    out_type=jax.ShapeDtypeStruct((NIDX, EMB), jnp.float32),
    mesh=_sc_mesh,
    scratch_types=[
        pltpu.VMEM((ROWS_PER_W // G, G), jnp.int32),
        pltpu.VMEM((ROWS_PER_W // G, G), jnp.int32),
        pltpu.VMEM((CHUNK, EMB), jnp.float32),
        pltpu.SemaphoreType.DMA,
    ],
    compiler_params=pltpu.CompilerParams(
        use_tc_tiling_on_sc=False, needs_layout_passes=False
    ),
)
def _sc_gather(table_hbm, idx_hbm, out_hbm, idx_raw, idx_v, rows_v, sem):
    wid = lax.axis_index("s") * NC + lax.axis_index("c")
    pltpu.sync_copy(idx_hbm.at[pl.ds(wid * (ROWS_PER_W // G), ROWS_PER_W // G)], idx_raw)

    lanes = lax.iota(jnp.int32, 16)

    def remap_body(t, _):
        u = t * 16 + lanes
        tc = lax.shift_right_logical(u, 11)
        k = jnp.bitwise_and(u, 2047)
        j_loc = lax.shift_right_logical(k, 2) * WIN + tc * 4 + jnp.bitwise_and(k, 3)
        v = plsc.load_gather(
            idx_raw, [lax.shift_right_logical(j_loc, 7), jnp.bitwise_and(j_loc, 127)]
        )
        j = jnp.bitwise_and(v, QSPLIT - 1) * 4 + lax.shift_right_logical(v, 18)
        idx_v[t // 8, pl.ds((t % 8) * 16, 16)] = j
        return ()

    lax.fori_loop(0, (ROWS_PER_W // G) * (G // 16), remap_body, (), unroll=False)

    def chunk_body(ci, _):
        copies = []
        for j in range(G_PER_CHUNK):
            cp = pltpu.make_async_copy(
                table_hbm.at[idx_v.at[ci * G_PER_CHUNK + j]],
                rows_v.at[pl.ds(j * G, G)],
                sem,
            )
            cp.start()
            copies.append(cp)
        for cp in copies:
            cp.wait()
        pltpu.sync_copy(rows_v, out_hbm.at[pl.ds(ci * (4 * BATCH) + wid * CHUNK, CHUNK)])
        return ()

    lax.fori_loop(0, N_CHUNK, chunk_body, (), unroll=False)


QSPLIT = 1 << 18
VPAD = 4 * QSPLIT
RB = 16384


def _transpose_body(t0_ref, t1_ref, t2_ref, t3_ref, o_ref):
    cat = jnp.concatenate(
        [t0_ref[...], t1_ref[...], t2_ref[...], t3_ref[...]], axis=0
    )
    o_ref[...] = jnp.transpose(cat)


def _relayout_table(tableT):
    grid = QSPLIT // RB

    last_blk = (VOCAB - 1) // RB

    def mk(q):
        return pl.BlockSpec(
            (EMB, RB),
            lambda i, q=q: (0, jnp.minimum((QSPLIT // RB) * q + i, last_blk)),
        )

    out = pl.pallas_call(
        _transpose_body,
        grid=(grid,),
        in_specs=[mk(0), mk(1), mk(2), mk(3)],
        out_specs=pl.BlockSpec((RB, 4 * EMB), lambda i: (i, 0)),
        out_shape=jax.ShapeDtypeStruct((QSPLIT, 4 * EMB), jnp.float32),
    )(tableT, tableT, tableT, tableT)
    return out.reshape(VPAD, EMB)


BLK = 2048


def _mlp_body(x0, x1, x2, x3, x4, w1_ref, b1_ref, w2_ref, b2_ref, w3_ref, b3_ref, o_ref):
    w1 = w1_ref[...]
    h = None
    for p, xp in enumerate((x0, x1, x2, x3, x4)):
        w1p = lax.slice(w1, (128 * p, 0), (128 * p + 128, H1))
        acc = jnp.dot(xp[...], w1p, preferred_element_type=jnp.float32)
        h = acc if h is None else h + acc
    h = jnp.maximum(h + b1_ref[...], 0.0)
    h = jnp.dot(h, w2_ref[...], preferred_element_type=jnp.float32)
    h = jnp.maximum(h + b2_ref[...], 0.0)
    o_ref[...] = jnp.dot(h, w3_ref[...], preferred_element_type=jnp.float32) + b3_ref[...]


def _mlp(h_stripes, W1, b1, W2, b2, W3, b3):
    in_dim = WIN * EMB

    def mk(p):
        return pl.BlockSpec((BLK, 128), lambda i, p=p: (p * (BATCH // BLK) + i, 0))

    return pl.pallas_call(
        _mlp_body,
        grid=(BATCH // BLK,),
        in_specs=[
            mk(0), mk(1), mk(2), mk(3), mk(4),
            pl.BlockSpec((in_dim, H1), lambda i: (0, 0)),
            pl.BlockSpec((1, H1), lambda i: (0, 0)),
            pl.BlockSpec((H1, H2), lambda i: (0, 0)),
            pl.BlockSpec((1, H2), lambda i: (0, 0)),
            pl.BlockSpec((H2, NCLS), lambda i: (0, 0)),
            pl.BlockSpec((1, NCLS), lambda i: (0, 0)),
        ],
        out_specs=pl.BlockSpec((BLK, NCLS), lambda i: (i, 0)),
        out_shape=jax.ShapeDtypeStruct((BATCH, NCLS), jnp.float32),
    )(h_stripes, h_stripes, h_stripes, h_stripes, h_stripes, W1, b1, W2, b2, W3, b3)


@jax.jit
def kernel(x, table, W1, b1, W2, b2, W3, b3):
    idx2d = x.astype(jnp.int32).reshape(NIDX // G, G)
    table_lin = _relayout_table(table.T)
    embeds = _sc_gather(table_lin, idx2d)
    h_tiles = embeds.reshape(NIDX * EMB // 128, 128)
    return _mlp(
        h_tiles,
        W1,
        b1.reshape(1, H1),
        W2,
        b2.reshape(1, H2),
        W3,
        b3.reshape(1, NCLS),
    )


# speedup vs baseline: 49.1586x; 1.0089x over previous
import functools

import jax
import jax.numpy as jnp
from jax import lax
from jax.experimental import pallas as pl
from jax.experimental.pallas import tpu as pltpu
from jax.experimental.pallas import tpu_sc as plsc

VOCAB = 1000000
EMB = 32
WIN = 20
H1 = 128
H2 = 64
NCLS = 9
BATCH = 16384
NIDX = BATCH * WIN

NC = 2
NS = 16
NW = NC * NS
HALF = BATCH // 2
NIDX_H = HALF * WIN
RPW = NIDX_H // NW
G = 128
CHUNK = 1024
G_PER_CHUNK = CHUNK // G
N_CHUNK = RPW // CHUNK
IDXROWS = RPW // G

_sc_mesh = plsc.VectorSubcoreMesh(
    core_axis_name="c", subcore_axis_name="s", num_cores=NC, num_subcores=NS
)


def _make_sc_gather(half):
    @functools.partial(
        pl.kernel,
        out_type=jax.ShapeDtypeStruct((NIDX_H, EMB), jnp.float32),
        mesh=_sc_mesh,
        scratch_types=[
            pltpu.VMEM((IDXROWS, G), jnp.int32),
            pltpu.VMEM((IDXROWS, G), jnp.int32),
            pltpu.VMEM((CHUNK, EMB), jnp.float32),
            pltpu.VMEM((CHUNK, EMB), jnp.float32),
            pltpu.SemaphoreType.DMA,
            pltpu.SemaphoreType.DMA,
        ],
        compiler_params=pltpu.CompilerParams(
            use_tc_tiling_on_sc=False, needs_layout_passes=False
        ),
    )
    def _g(table_hbm, idx_hbm, out_hbm, idx_raw, idx_v, rows_a, rows_b, sem_a, sem_b):
        wid = lax.axis_index("s") * NC + lax.axis_index("c")
        pltpu.sync_copy(
            idx_hbm.at[pl.ds(half * (NIDX_H // G) + wid * IDXROWS, IDXROWS)], idx_raw
        )

        lanes = lax.iota(jnp.int32, 16)

        def remap_body(t, _):
            u = t * 16 + lanes
            tc = lax.shift_right_logical(u, 10)
            k = jnp.bitwise_and(u, 1023)
            j_loc = lax.shift_right_logical(k, 2) * WIN + tc * 4 + jnp.bitwise_and(k, 3)
            v = plsc.load_gather(
                idx_raw, [lax.shift_right_logical(j_loc, 7), jnp.bitwise_and(j_loc, 127)]
            )
            j = jnp.bitwise_and(v, QSPLIT - 1) * 4 + lax.shift_right_logical(v, 18)
            idx_v[t // 8, pl.ds((t % 8) * 16, 16)] = j
            return ()

        lax.fori_loop(0, RPW // 16, remap_body, (), unroll=False)

        def mk_copies(ci, buf, sem):
            return [
                pltpu.make_async_copy(
                    table_hbm.at[idx_v.at[ci * G_PER_CHUNK + g]],
                    buf.at[pl.ds(g * G, G)],
                    sem,
                )
                for g in range(G_PER_CHUNK)
            ]

        def fire(ci, buf, sem):
            for cp in mk_copies(ci, buf, sem):
                cp.start()

        def drain_out(ci, buf, sem):
            for cp in mk_copies(ci, buf, sem):
                cp.wait()
            pltpu.sync_copy(
                buf, out_hbm.at[pl.ds(ci * (4 * HALF) + wid * CHUNK, CHUNK)]
            )

        fire(0, rows_a, sem_a)
        fire(1, rows_b, sem_b)

        def pipe_body(c2, _):
            even = c2 * 2
            drain_out(even, rows_a, sem_a)
            fire(even + 2, rows_a, sem_a)
            drain_out(even + 1, rows_b, sem_b)

            @pl.when(c2 < N_CHUNK // 2 - 1)
            def _():
                fire(even + 3, rows_b, sem_b)

            return ()

        lax.fori_loop(0, N_CHUNK // 2, pipe_body, (), unroll=False)
        drain_out(N_CHUNK - 1, rows_a, sem_a)

    return _g


_sc_gather0 = _make_sc_gather(0)
_sc_gather1 = _make_sc_gather(1)


QSPLIT = 1 << 18
VPAD = 4 * QSPLIT
RB = 16384


def _transpose_body(t0_ref, t1_ref, t2_ref, t3_ref, o_ref):
    cat = jnp.concatenate(
        [t0_ref[...], t1_ref[...], t2_ref[...], t3_ref[...]], axis=0
    )
    o_ref[...] = jnp.transpose(cat)


def _relayout_table(tableT):
    grid = QSPLIT // RB

    last_blk = (VOCAB - 1) // RB

    def mk(q):
        return pl.BlockSpec(
            (EMB, RB),
            lambda i, q=q: (0, jnp.minimum((QSPLIT // RB) * q + i, last_blk)),
        )

    out = pl.pallas_call(
        _transpose_body,
        grid=(grid,),
        in_specs=[mk(0), mk(1), mk(2), mk(3)],
        out_specs=pl.BlockSpec((RB, 4 * EMB), lambda i: (i, 0)),
        out_shape=jax.ShapeDtypeStruct((QSPLIT, 4 * EMB), jnp.float32),
    )(tableT, tableT, tableT, tableT)
    return out.reshape(VPAD, EMB)


BLK = 2048


def _mlp_body(x0, x1, x2, x3, x4, w1_ref, b1_ref, w2_ref, b2_ref, w3_ref, b3_ref, o_ref):
    w1 = w1_ref[...]
    h = None
    for p, xp in enumerate((x0, x1, x2, x3, x4)):
        w1p = lax.slice(w1, (128 * p, 0), (128 * p + 128, H1))
        acc = jnp.dot(xp[...], w1p, preferred_element_type=jnp.float32)
        h = acc if h is None else h + acc
    h = jnp.maximum(h + b1_ref[...], 0.0)
    h = jnp.dot(h, w2_ref[...], preferred_element_type=jnp.float32)
    h = jnp.maximum(h + b2_ref[...], 0.0)
    o_ref[...] = jnp.dot(h, w3_ref[...], preferred_element_type=jnp.float32) + b3_ref[...]


def _mlp(h_stripes, W1, b1, W2, b2, W3, b3):
    in_dim = WIN * EMB

    def mk(p):
        return pl.BlockSpec((BLK, 128), lambda i, p=p: (p * (HALF // BLK) + i, 0))

    return pl.pallas_call(
        _mlp_body,
        grid=(HALF // BLK,),
        in_specs=[
            mk(0), mk(1), mk(2), mk(3), mk(4),
            pl.BlockSpec((in_dim, H1), lambda i: (0, 0)),
            pl.BlockSpec((1, H1), lambda i: (0, 0)),
            pl.BlockSpec((H1, H2), lambda i: (0, 0)),
            pl.BlockSpec((1, H2), lambda i: (0, 0)),
            pl.BlockSpec((H2, NCLS), lambda i: (0, 0)),
            pl.BlockSpec((1, NCLS), lambda i: (0, 0)),
        ],
        out_specs=pl.BlockSpec((BLK, NCLS), lambda i: (i, 0)),
        out_shape=jax.ShapeDtypeStruct((HALF, NCLS), jnp.float32),
    )(h_stripes, h_stripes, h_stripes, h_stripes, h_stripes, W1, b1, W2, b2, W3, b3)


@jax.jit
def kernel(x, table, W1, b1, W2, b2, W3, b3):
    idx2d = x.astype(jnp.int32).reshape(NIDX // G, G)
    table_lin = _relayout_table(table.T)
    b1r, b2r, b3r = b1.reshape(1, H1), b2.reshape(1, H2), b3.reshape(1, NCLS)
    e0 = _sc_gather0(table_lin, idx2d)
    e1 = _sc_gather1(table_lin, idx2d)
    o0 = _mlp(e0.reshape(NIDX_H * EMB // 128, 128), W1, b1r, W2, b2r, W3, b3r)
    o1 = _mlp(e1.reshape(NIDX_H * EMB // 128, 128), W1, b1r, W2, b2r, W3, b3r)
    return jnp.concatenate([o0, o1], axis=0)
